# initial kernel scaffold (unmeasured)
import jax
import jax.numpy as jnp
from jax import lax
from jax.experimental import pallas as pl
from jax.experimental.pallas import tpu as pltpu

N_DEV = 4
E_LOCAL = 4


def kernel(x, router_W, route_idx, expert_W):
    n_tok, d = x.shape
    _, h = expert_W.shape[1:]

    def body(x_ref, rw_ref, idx_ref, ew_ref, out_ref, comm_ref,
             send_sems, recv_sems):
        my = lax.axis_index("i")
        left = lax.rem(my + N_DEV - 1, N_DEV)
        right = lax.rem(my + 1, N_DEV)

        barrier_sem = pltpu.get_barrier_semaphore()
        for nbr in (left, right):
            pl.semaphore_signal(
                barrier_sem, inc=1,
                device_id=(nbr,), device_id_type=pl.DeviceIdType.MESH,
            )
        pl.semaphore_wait(barrier_sem, 2)

        xv = x_ref[...]
        scores = jnp.dot(xv, rw_ref[...],
                         preferred_element_type=jnp.float32)
        s_max = jnp.max(scores, axis=-1, keepdims=True)
        probs = jnp.exp(scores - s_max)
        probs = probs / jnp.sum(probs, axis=-1, keepdims=True)

        e_ids = lax.broadcasted_iota(jnp.int32, scores.shape, 1)
        idx0 = idx_ref[:, 0:1]
        idx1 = idx_ref[:, 1:2]
        top2 = (e_ids == idx0) | (e_ids == idx1)
        w = jnp.where(top2, probs, 0.0)
        w = w / jnp.sum(w, axis=-1, keepdims=True)

        w_local = lax.dynamic_slice(w, (0, my * E_LOCAL), (n_tok, E_LOCAL))

        acc = jnp.zeros((n_tok, h), dtype=jnp.float32)
        for j in range(E_LOCAL):
            gx = (w_local[:, j:j + 1] * xv).astype(jnp.bfloat16)
            wj = ew_ref[j].astype(jnp.bfloat16)
            acc = acc + jnp.dot(gx, wj, preferred_element_type=jnp.float32)

        comm_ref[0] = acc

        for hhop in range(N_DEV - 1):
            rdma = pltpu.make_async_remote_copy(
                src_ref=comm_ref.at[hhop],
                dst_ref=comm_ref.at[hhop + 1],
                send_sem=send_sems.at[hhop],
                recv_sem=recv_sems.at[hhop],
                device_id=(right,),
                device_id_type=pl.DeviceIdType.MESH,
            )
            rdma.start()
            rdma.wait()
            acc = acc + comm_ref[hhop + 1]

        out_ref[...] = acc

    return pl.pallas_call(
        body,
        out_shape=jax.ShapeDtypeStruct((n_tok, h), jnp.float32),
        in_specs=[pl.BlockSpec(memory_space=pltpu.VMEM)] * 4,
        out_specs=pl.BlockSpec(memory_space=pltpu.VMEM),
        scratch_shapes=[
            pltpu.VMEM((N_DEV, n_tok, h), jnp.float32),
            pltpu.SemaphoreType.DMA((N_DEV - 1,)),
            pltpu.SemaphoreType.DMA((N_DEV - 1,)),
        ],
        compiler_params=pltpu.CompilerParams(collective_id=0),
    )(x, router_W, route_idx, expert_W)


# baseline (device time: 84756 ns/iter reference)
import jax
import jax.numpy as jnp
from jax import lax
from jax.experimental import pallas as pl
from jax.experimental.pallas import tpu as pltpu

N_DEV = 4
E_LOCAL = 4


def kernel(x, router_W, route_idx, expert_W):
    n_tok, d = x.shape
    _, h = expert_W.shape[1:]

    def body(x_ref, rw_ref, idx_ref, ew_ref, out_ref, comm_ref,
             send_sems, recv_sems):
        my = lax.axis_index("i")
        left = lax.rem(my + N_DEV - 1, N_DEV)
        right = lax.rem(my + 1, N_DEV)

        barrier_sem = pltpu.get_barrier_semaphore()
        for nbr in (left, right):
            pl.semaphore_signal(
                barrier_sem, inc=1,
                device_id=(nbr,), device_id_type=pl.DeviceIdType.MESH,
            )
        pl.semaphore_wait(barrier_sem, 2)

        xv = x_ref[...]
        scores = jnp.dot(xv, rw_ref[...],
                         preferred_element_type=jnp.float32)
        s_max = jnp.max(scores, axis=-1, keepdims=True)
        probs = jnp.exp(scores - s_max)
        probs = probs / jnp.sum(probs, axis=-1, keepdims=True)

        e_ids = lax.broadcasted_iota(jnp.int32, scores.shape, 1)
        idx0 = idx_ref[:, 0:1]
        idx1 = idx_ref[:, 1:2]
        top2 = (e_ids == idx0) | (e_ids == idx1)
        w = jnp.where(top2, probs, 0.0)
        w = w / jnp.sum(w, axis=-1, keepdims=True)

        n_exp = w.shape[1]
        rows = lax.broadcasted_iota(jnp.int32, (n_exp, E_LOCAL), 0)
        cols = lax.broadcasted_iota(jnp.int32, (n_exp, E_LOCAL), 1)
        sel = (rows == my * E_LOCAL + cols).astype(jnp.float32)
        w_local = jnp.dot(w, sel, preferred_element_type=jnp.float32)

        acc = jnp.zeros((n_tok, h), dtype=jnp.float32)
        for j in range(E_LOCAL):
            gx = (w_local[:, j:j + 1] * xv).astype(jnp.bfloat16)
            wj = ew_ref[j].astype(jnp.bfloat16)
            acc = acc + jnp.dot(gx, wj, preferred_element_type=jnp.float32)

        comm_ref[0] = acc

        for hhop in range(N_DEV - 1):
            rdma = pltpu.make_async_remote_copy(
                src_ref=comm_ref.at[hhop],
                dst_ref=comm_ref.at[hhop + 1],
                send_sem=send_sems.at[hhop],
                recv_sem=recv_sems.at[hhop],
                device_id=(right,),
                device_id_type=pl.DeviceIdType.MESH,
            )
            rdma.start()
            rdma.wait()
            acc = acc + comm_ref[hhop + 1]

        out_ref[...] = acc

    return pl.pallas_call(
        body,
        out_shape=jax.ShapeDtypeStruct((n_tok, h), jnp.float32),
        in_specs=[pl.BlockSpec(memory_space=pltpu.VMEM)] * 4,
        out_specs=pl.BlockSpec(memory_space=pltpu.VMEM),
        scratch_shapes=[
            pltpu.VMEM((N_DEV, n_tok, h), jnp.float32),
            pltpu.SemaphoreType.DMA((N_DEV - 1,)),
            pltpu.SemaphoreType.DMA((N_DEV - 1,)),
        ],
        compiler_params=pltpu.CompilerParams(collective_id=0),
    )(x, router_W, route_idx, expert_W)


# device time: 37552 ns/iter; 2.2570x vs baseline; 2.2570x over previous
import jax
import jax.numpy as jnp
from jax import lax
from jax.experimental import pallas as pl
from jax.experimental.pallas import tpu as pltpu

N_DEV = 4
E_LOCAL = 4


def kernel(x, router_W, route_idx, expert_W):
    n_tok, d = x.shape
    _, h = expert_W.shape[1:]

    def body(x_ref, rw_ref, idx_ref, ew_ref, out_ref, sbuf, rbuf,
             send_sems, recv_sems):
        my = lax.axis_index("i")
        p1 = my ^ 1
        p2 = 3 - my

        barrier_sem = pltpu.get_barrier_semaphore()
        for nbr in (p1, p2):
            pl.semaphore_signal(
                barrier_sem, inc=1,
                device_id=(nbr,), device_id_type=pl.DeviceIdType.MESH,
            )
        pl.semaphore_wait(barrier_sem, 2)

        xv = x_ref[...]
        scores = jnp.dot(xv, rw_ref[...],
                         preferred_element_type=jnp.float32)
        s_max = jnp.max(scores, axis=-1, keepdims=True)
        probs = jnp.exp(scores - s_max)
        probs = probs / jnp.sum(probs, axis=-1, keepdims=True)

        e_ids = lax.broadcasted_iota(jnp.int32, scores.shape, 1)
        idx0 = idx_ref[:, 0:1]
        idx1 = idx_ref[:, 1:2]
        top2 = (e_ids == idx0) | (e_ids == idx1)
        w = jnp.where(top2, probs, 0.0)
        w = w / jnp.sum(w, axis=-1, keepdims=True)

        n_exp = w.shape[1]
        rows = lax.broadcasted_iota(jnp.int32, (n_exp, E_LOCAL), 0)
        cols = lax.broadcasted_iota(jnp.int32, (n_exp, E_LOCAL), 1)
        sel = (rows == my * E_LOCAL + cols).astype(jnp.float32)
        w_local = jnp.dot(w, sel, preferred_element_type=jnp.float32)

        acc = jnp.zeros((n_tok, h), dtype=jnp.float32)
        for j in range(E_LOCAL):
            gx = (w_local[:, j:j + 1] * xv).astype(jnp.bfloat16)
            wj = ew_ref[j].astype(jnp.bfloat16)
            acc = acc + jnp.dot(gx, wj, preferred_element_type=jnp.float32)

        for stage, partner in ((0, p1), (1, p2)):
            sbuf[stage] = acc.astype(jnp.bfloat16)
            rdma = pltpu.make_async_remote_copy(
                src_ref=sbuf.at[stage],
                dst_ref=rbuf.at[stage],
                send_sem=send_sems.at[stage],
                recv_sem=recv_sems.at[stage],
                device_id=(partner,),
                device_id_type=pl.DeviceIdType.MESH,
            )
            rdma.start()
            rdma.wait()
            acc = acc + rbuf[stage].astype(jnp.float32)

        out_ref[...] = acc

    return pl.pallas_call(
        body,
        out_shape=jax.ShapeDtypeStruct((n_tok, h), jnp.float32),
        in_specs=[pl.BlockSpec(memory_space=pltpu.VMEM)] * 4,
        out_specs=pl.BlockSpec(memory_space=pltpu.VMEM),
        scratch_shapes=[
            pltpu.VMEM((2, n_tok, h), jnp.bfloat16),
            pltpu.VMEM((2, n_tok, h), jnp.bfloat16),
            pltpu.SemaphoreType.DMA((2,)),
            pltpu.SemaphoreType.DMA((2,)),
        ],
        compiler_params=pltpu.CompilerParams(collective_id=0),
    )(x, router_W, route_idx, expert_W)


# device time: 28743 ns/iter; 2.9488x vs baseline; 1.3065x over previous
import jax
import jax.numpy as jnp
from jax import lax
from jax.experimental import pallas as pl
from jax.experimental.pallas import tpu as pltpu

N_DEV = 4
E_LOCAL = 4


def kernel(x, router_W, route_idx, expert_W):
    n_tok, d = x.shape
    _, h = expert_W.shape[1:]

    C = 4
    rows = n_tok // C

    def body(x_ref, rw_ref, idx_ref, ew_ref, out_ref, sbuf0, rbuf0,
             sbuf1, rbuf1, send_sems, recv_sems):
        my = lax.axis_index("i")
        p1 = my ^ 1
        p2 = 3 - my

        barrier_sem = pltpu.get_barrier_semaphore()
        for nbr in (p1, p2):
            pl.semaphore_signal(
                barrier_sem, inc=1,
                device_id=(nbr,), device_id_type=pl.DeviceIdType.MESH,
            )
        pl.semaphore_wait(barrier_sem, 2)

        xv = x_ref[...]
        scores = jnp.dot(xv, rw_ref[...],
                         preferred_element_type=jnp.float32)
        s_max = jnp.max(scores, axis=-1, keepdims=True)
        probs = jnp.exp(scores - s_max)
        probs = probs / jnp.sum(probs, axis=-1, keepdims=True)

        e_ids = lax.broadcasted_iota(jnp.int32, scores.shape, 1)
        idx0 = idx_ref[:, 0:1]
        idx1 = idx_ref[:, 1:2]
        top2 = (e_ids == idx0) | (e_ids == idx1)
        w = jnp.where(top2, probs, 0.0)
        w = w / jnp.sum(w, axis=-1, keepdims=True)

        n_exp = w.shape[1]
        sel_r = lax.broadcasted_iota(jnp.int32, (n_exp, E_LOCAL), 0)
        sel_c = lax.broadcasted_iota(jnp.int32, (n_exp, E_LOCAL), 1)
        sel = (sel_r == my * E_LOCAL + sel_c).astype(jnp.float32)
        w_local = jnp.dot(w, sel, preferred_element_type=jnp.float32)

        acc = jnp.zeros((n_tok, h), dtype=jnp.float32)
        for j in range(E_LOCAL):
            gx = (w_local[:, j:j + 1] * xv).astype(jnp.bfloat16)
            wj = ew_ref[j].astype(jnp.bfloat16)
            acc = acc + jnp.dot(gx, wj, preferred_element_type=jnp.float32)

        def mk(stage, c, partner):
            sb, rb = (sbuf0, rbuf0) if stage == 0 else (sbuf1, rbuf1)
            return pltpu.make_async_remote_copy(
                src_ref=sb.at[c],
                dst_ref=rb.at[c],
                send_sem=send_sems.at[stage * C + c],
                recv_sem=recv_sems.at[stage * C + c],
                device_id=(partner,),
                device_id_type=pl.DeviceIdType.MESH,
            )

        for c in range(C):
            sbuf0[c] = acc[c * rows:(c + 1) * rows].astype(jnp.bfloat16)
            mk(0, c, p1).start()

        acc2 = []
        for c in range(C):
            mk(0, c, p1).wait_recv()
            a2 = acc[c * rows:(c + 1) * rows] + rbuf0[c].astype(jnp.float32)
            acc2.append(a2)
            sbuf1[c] = a2.astype(jnp.bfloat16)
            mk(1, c, p2).start()

        for c in range(C):
            mk(1, c, p2).wait()
            out_ref[c * rows:(c + 1) * rows, :] = (
                acc2[c] + rbuf1[c].astype(jnp.float32))
        for c in range(C):
            mk(0, c, p1).wait_send()

    return pl.pallas_call(
        body,
        out_shape=jax.ShapeDtypeStruct((n_tok, h), jnp.float32),
        in_specs=[pl.BlockSpec(memory_space=pltpu.VMEM)] * 4,
        out_specs=pl.BlockSpec(memory_space=pltpu.VMEM),
        scratch_shapes=[
            pltpu.VMEM((C, n_tok // C, h), jnp.bfloat16),
            pltpu.VMEM((C, n_tok // C, h), jnp.bfloat16),
            pltpu.VMEM((C, n_tok // C, h), jnp.bfloat16),
            pltpu.VMEM((C, n_tok // C, h), jnp.bfloat16),
            pltpu.SemaphoreType.DMA((2 * C,)),
            pltpu.SemaphoreType.DMA((2 * C,)),
        ],
        compiler_params=pltpu.CompilerParams(collective_id=0),
    )(x, router_W, route_idx, expert_W)


# device time: 28628 ns/iter; 2.9606x vs baseline; 1.0040x over previous
import jax
import jax.numpy as jnp
from jax import lax
from jax.experimental import pallas as pl
from jax.experimental.pallas import tpu as pltpu

N_DEV = 4
E_LOCAL = 4


def kernel(x, router_W, route_idx, expert_W):
    n_tok, d = x.shape
    _, h = expert_W.shape[1:]

    C = 4
    rows = n_tok // C

    def body(x_ref, rw_ref, idx_ref, ew_ref, out_ref, sbuf0, rbuf0,
             sbuf1, rbuf1, send_sems, recv_sems):
        my = lax.axis_index("i")
        p1 = my ^ 1
        p2 = 3 - my

        barrier_sem = pltpu.get_barrier_semaphore()
        for nbr in (p1, p2):
            pl.semaphore_signal(
                barrier_sem, inc=1,
                device_id=(nbr,), device_id_type=pl.DeviceIdType.MESH,
            )
        pl.semaphore_wait(barrier_sem, 2)

        xv = x_ref[...]
        scores = jnp.dot(xv, rw_ref[...],
                         preferred_element_type=jnp.float32)
        s_max = jnp.max(scores, axis=-1, keepdims=True)
        probs = jnp.exp(scores - s_max)
        probs = probs / jnp.sum(probs, axis=-1, keepdims=True)

        e_ids = lax.broadcasted_iota(jnp.int32, scores.shape, 1)
        idx0 = idx_ref[:, 0:1]
        idx1 = idx_ref[:, 1:2]
        top2 = (e_ids == idx0) | (e_ids == idx1)
        w = jnp.where(top2, probs, 0.0)
        w = w / jnp.sum(w, axis=-1, keepdims=True)

        n_exp = w.shape[1]
        sel_r = lax.broadcasted_iota(jnp.int32, (n_exp, E_LOCAL), 0)
        sel_c = lax.broadcasted_iota(jnp.int32, (n_exp, E_LOCAL), 1)
        sel = (sel_r == my * E_LOCAL + sel_c).astype(jnp.float32)
        w_local = jnp.dot(w, sel,
                          preferred_element_type=jnp.float32
                          ).astype(jnp.bfloat16)
        xb = xv.astype(jnp.bfloat16)
        wjs = [ew_ref[j].astype(jnp.bfloat16) for j in range(E_LOCAL)]

        def mk(stage, c, partner):
            sb, rb = (sbuf0, rbuf0) if stage == 0 else (sbuf1, rbuf1)
            return pltpu.make_async_remote_copy(
                src_ref=sb.at[c],
                dst_ref=rb.at[c],
                send_sem=send_sems.at[stage * C + c],
                recv_sem=recv_sems.at[stage * C + c],
                device_id=(partner,),
                device_id_type=pl.DeviceIdType.MESH,
            )

        accs = []
        for c in range(C):
            lo = c * rows
            acc_c = jnp.zeros((rows, h), dtype=jnp.float32)
            for j in range(E_LOCAL):
                gx = w_local[lo:lo + rows, j:j + 1] * xb[lo:lo + rows]
                acc_c = acc_c + jnp.dot(gx, wjs[j],
                                        preferred_element_type=jnp.float32)
            accs.append(acc_c)
            sbuf0[c] = acc_c.astype(jnp.bfloat16)
            mk(0, c, p1).start()

        acc2 = []
        for c in range(C):
            mk(0, c, p1).wait_recv()
            a2 = accs[c] + rbuf0[c].astype(jnp.float32)
            acc2.append(a2)
            sbuf1[c] = a2.astype(jnp.bfloat16)
            mk(1, c, p2).start()

        for c in range(C):
            mk(1, c, p2).wait()
            out_ref[c * rows:(c + 1) * rows, :] = (
                acc2[c] + rbuf1[c].astype(jnp.float32))
        for c in range(C):
            mk(0, c, p1).wait_send()

    return pl.pallas_call(
        body,
        out_shape=jax.ShapeDtypeStruct((n_tok, h), jnp.float32),
        in_specs=[pl.BlockSpec(memory_space=pltpu.VMEM)] * 4,
        out_specs=pl.BlockSpec(memory_space=pltpu.VMEM),
        scratch_shapes=[
            pltpu.VMEM((C, n_tok // C, h), jnp.bfloat16),
            pltpu.VMEM((C, n_tok // C, h), jnp.bfloat16),
            pltpu.VMEM((C, n_tok // C, h), jnp.bfloat16),
            pltpu.VMEM((C, n_tok // C, h), jnp.bfloat16),
            pltpu.SemaphoreType.DMA((2 * C,)),
            pltpu.SemaphoreType.DMA((2 * C,)),
        ],
        compiler_params=pltpu.CompilerParams(collective_id=0),
    )(x, router_W, route_idx, expert_W)


# device time: 27383 ns/iter; 3.0952x vs baseline; 1.0455x over previous
import jax
import jax.numpy as jnp
from jax import lax
from jax.experimental import pallas as pl
from jax.experimental.pallas import tpu as pltpu

N_DEV = 4
E_LOCAL = 4


def kernel(x, router_W, route_idx, expert_W):
    n_tok, d = x.shape
    _, h = expert_W.shape[1:]
    rows = n_tok // N_DEV

    def body(x_ref, rw_ref, idx_ref, ew_ref, out_ref, pbuf, red_rbuf,
             bc_sbuf, bc_rbuf, red_ssems, red_rsems, bc_ssems, bc_rsems):
        my = lax.axis_index("i")

        barrier_sem = pltpu.get_barrier_semaphore()
        for t in range(1, N_DEV):
            pl.semaphore_signal(
                barrier_sem, inc=1,
                device_id=(lax.rem(my + t, N_DEV),),
                device_id_type=pl.DeviceIdType.MESH,
            )
        pl.semaphore_wait(barrier_sem, N_DEV - 1)

        xv = x_ref[...]
        scores = jnp.dot(xv, rw_ref[...],
                         preferred_element_type=jnp.float32)
        s_max = jnp.max(scores, axis=-1, keepdims=True)
        probs = jnp.exp(scores - s_max)
        probs = probs / jnp.sum(probs, axis=-1, keepdims=True)

        e_ids = lax.broadcasted_iota(jnp.int32, scores.shape, 1)
        top2 = (e_ids == idx_ref[:, 0:1]) | (e_ids == idx_ref[:, 1:2])
        w = jnp.where(top2, probs, 0.0)
        w = w / jnp.sum(w, axis=-1, keepdims=True)

        n_exp = w.shape[1]
        sel_r = lax.broadcasted_iota(jnp.int32, (n_exp, E_LOCAL), 0)
        sel_c = lax.broadcasted_iota(jnp.int32, (n_exp, E_LOCAL), 1)
        sel = (sel_r == my * E_LOCAL + sel_c).astype(jnp.float32)
        w_local = jnp.dot(w, sel,
                          preferred_element_type=jnp.float32
                          ).astype(jnp.bfloat16)
        xb = xv.astype(jnp.bfloat16)
        wjs = [ew_ref[j].astype(jnp.bfloat16) for j in range(E_LOCAL)]

        def rdma(src, dst, ssem, rsem, target):
            return pltpu.make_async_remote_copy(
                src_ref=src, dst_ref=dst, send_sem=ssem, recv_sem=rsem,
                device_id=(target,), device_id_type=pl.DeviceIdType.MESH,
            )

        for k in range(N_DEV):
            lo = k * rows
            acc_c = jnp.zeros((rows, h), dtype=jnp.float32)
            for j in range(E_LOCAL):
                gx = w_local[lo:lo + rows, j:j + 1] * xb[lo:lo + rows]
                acc_c = acc_c + jnp.dot(
                    gx, wjs[j], preferred_element_type=jnp.float32)
            pbuf[k] = acc_c.astype(jnp.bfloat16)

        red = []
        for t in range(1, N_DEV):
            dst_dev = lax.rem(my + t, N_DEV)
            r = rdma(pbuf.at[dst_dev], red_rbuf.at[3 - t],
                     red_ssems.at[t - 1], red_rsems.at[3 - t], dst_dev)
            r.start()
            red.append(r)

        acc_my = pbuf[my].astype(jnp.float32)
        for t in range(1, N_DEV):
            rr = rdma(pbuf.at[0], red_rbuf.at[t - 1],
                      red_ssems.at[0], red_rsems.at[t - 1], my)
            rr.wait_recv()
            acc_my = acc_my + red_rbuf[t - 1].astype(jnp.float32)

        out_ref[pl.ds(my * rows, rows), :] = acc_my
        bc_sbuf[...] = acc_my.astype(jnp.bfloat16)

        bc = []
        for t in range(1, N_DEV):
            dst_dev = lax.rem(my + t, N_DEV)
            b = rdma(bc_sbuf, bc_rbuf.at[3 - t],
                     bc_ssems.at[t - 1], bc_rsems.at[3 - t], dst_dev)
            b.start()
            bc.append(b)

        for t in range(1, N_DEV):
            origin = lax.rem(my + t, N_DEV)
            br = rdma(bc_sbuf, bc_rbuf.at[t - 1],
                      bc_ssems.at[0], bc_rsems.at[t - 1], my)
            br.wait_recv()
            out_ref[pl.ds(origin * rows, rows), :] = (
                bc_rbuf[t - 1].astype(jnp.float32))

        for r in red:
            r.wait_send()
        for b in bc:
            b.wait_send()

    return pl.pallas_call(
        body,
        out_shape=jax.ShapeDtypeStruct((n_tok, h), jnp.float32),
        in_specs=[pl.BlockSpec(memory_space=pltpu.VMEM)] * 4,
        out_specs=pl.BlockSpec(memory_space=pltpu.VMEM),
        scratch_shapes=[
            pltpu.VMEM((N_DEV, rows, h), jnp.bfloat16),
            pltpu.VMEM((N_DEV - 1, rows, h), jnp.bfloat16),
            pltpu.VMEM((rows, h), jnp.bfloat16),
            pltpu.VMEM((N_DEV - 1, rows, h), jnp.bfloat16),
            pltpu.SemaphoreType.DMA((N_DEV - 1,)),
            pltpu.SemaphoreType.DMA((N_DEV - 1,)),
            pltpu.SemaphoreType.DMA((N_DEV - 1,)),
            pltpu.SemaphoreType.DMA((N_DEV - 1,)),
        ],
        compiler_params=pltpu.CompilerParams(collective_id=0),
    )(x, router_W, route_idx, expert_W)
